# baseline (device time: 60377 ns/iter reference)
import jax
import jax.numpy as jnp
from jax import lax
from jax.experimental import pallas as pl
from jax.experimental.pallas import tpu as pltpu

N_EXPERTS = 4
EXPERTS_PER_SHARD = 2
CAP = 320


def kernel(x, assign, W1, W2):
    tokens, d_model = x.shape
    my_x = lax.axis_index("x")

    oh = (assign[:, None] == jnp.arange(N_EXPERTS, dtype=assign.dtype)[None, :]).astype(jnp.int32)
    pos = ((jnp.cumsum(oh, axis=0) - 1) * oh).sum(axis=1)
    ap_row = jnp.stack([assign.astype(jnp.int32), pos])
    ap_col = ap_row.T

    f32 = jnp.float32
    bf16 = jnp.bfloat16

    def body(x_ref, apr_ref, apc_ref, w1_ref, w2_ref,
             out_ref, xin, sendbuf, rescat, xcat, send_sems, recv_sems):
        mx = lax.axis_index("x")
        my = lax.axis_index("y")
        mz = lax.axis_index("z")
        peer = (1 - mx, my, mz)
        e_loc = [2 * mx + k for k in range(EXPERTS_PER_SHARD)]
        e_out = [2 * (1 - mx) + k for k in range(EXPERTS_PER_SHARD)]

        def make_p(e):
            iota = lax.broadcasted_iota(jnp.int32, (CAP, tokens), 0)
            sel = (apr_ref[0:1, :] == e) & (apr_ref[1:2, :] == iota)
            return sel.astype(bf16)

        def make_p2(e0, e1):
            iota = lax.broadcasted_iota(jnp.int32, (2 * CAP, tokens), 0)
            low = iota < CAP
            expert = jnp.where(low, e0, e1)
            rank = jnp.where(low, iota, iota - CAP)
            sel = (apr_ref[0:1, :] == expert) & (apr_ref[1:2, :] == rank)
            return sel.astype(bf16)

        def make_pt_all():
            c = lax.broadcasted_iota(jnp.int32, (tokens, 4 * CAP), 1)
            expert = jnp.where(
                c < CAP, e_loc[0],
                jnp.where(c < 2 * CAP, e_loc[1],
                          jnp.where(c < 3 * CAP, e_out[0], e_out[1])))
            rank = jnp.where(
                c < CAP, c,
                jnp.where(c < 2 * CAP, c - CAP,
                          jnp.where(c < 3 * CAP, c - 2 * CAP, c - 3 * CAP)))
            sel = (apc_ref[:, 0:1] == expert) & (apc_ref[:, 1:2] == rank)
            return sel.astype(bf16)

        barrier_sem = pltpu.get_barrier_semaphore()
        pl.semaphore_signal(barrier_sem, inc=1, device_id=peer,
                            device_id_type=pl.DeviceIdType.MESH)
        pl.semaphore_wait(barrier_sem, 1)

        xb = x_ref[...].astype(bf16)

        sendbuf[...] = jnp.dot(make_p2(e_out[0], e_out[1]), xb,
                               preferred_element_type=f32).astype(bf16)
        rdma_x = []
        for k in range(EXPERTS_PER_SHARD):
            r = pltpu.make_async_remote_copy(
                src_ref=sendbuf.at[pl.ds(k * CAP, CAP), :],
                dst_ref=xin.at[pl.ds(k * CAP, CAP), :],
                send_sem=send_sems.at[k], recv_sem=recv_sems.at[k],
                device_id=peer, device_id_type=pl.DeviceIdType.MESH)
            r.start()
            rdma_x.append(r)

        rdma_r = []
        for k in range(EXPERTS_PER_SHARD):
            xcat[pl.ds(0, CAP), :] = jnp.dot(make_p(e_loc[k]), xb,
                                             preferred_element_type=f32)
            rdma_x[k].wait_recv()
            xcat[pl.ds(CAP, CAP), :] = xin[pl.ds(k * CAP, CAP), :].astype(f32)
            h = jnp.maximum(jnp.dot(xcat[...], w1_ref[k], preferred_element_type=f32), 0.0)
            res = jnp.dot(h, w2_ref[k], preferred_element_type=f32)
            rescat[pl.ds(k * CAP, CAP), :] = res[0:CAP].astype(bf16)
            rdma_x[k].wait_send()
            sendbuf[pl.ds(k * CAP, CAP), :] = res[CAP:2 * CAP].astype(bf16)
            r = pltpu.make_async_remote_copy(
                src_ref=sendbuf.at[pl.ds(k * CAP, CAP), :],
                dst_ref=rescat.at[pl.ds((2 + k) * CAP, CAP), :],
                send_sem=send_sems.at[EXPERTS_PER_SHARD + k],
                recv_sem=recv_sems.at[EXPERTS_PER_SHARD + k],
                device_id=peer, device_id_type=pl.DeviceIdType.MESH)
            r.start()
            rdma_r.append(r)

        for r in rdma_r:
            r.wait_recv()
        out_ref[...] = jnp.dot(make_pt_all(), rescat[...],
                               preferred_element_type=f32)
        for r in rdma_r:
            r.wait_send()

    return pl.pallas_call(
        body,
        out_shape=jax.ShapeDtypeStruct((tokens, d_model), x.dtype),
        in_specs=[pl.BlockSpec(memory_space=pltpu.VMEM)] * 5,
        out_specs=pl.BlockSpec(memory_space=pltpu.VMEM),
        scratch_shapes=[
            pltpu.VMEM((2 * CAP, d_model), bf16),
            pltpu.VMEM((2 * CAP, d_model), bf16),
            pltpu.VMEM((4 * CAP, d_model), bf16),
            pltpu.VMEM((2 * CAP, d_model), f32),
            pltpu.SemaphoreType.DMA((2 * EXPERTS_PER_SHARD,)),
            pltpu.SemaphoreType.DMA((2 * EXPERTS_PER_SHARD,)),
        ],
        compiler_params=pltpu.CompilerParams(
            collective_id=0,
            vmem_limit_bytes=60 * 1024 * 1024,
        ),
    )(x, ap_row, ap_col, W1, W2)


# device time: 60125 ns/iter; 1.0042x vs baseline; 1.0042x over previous
import jax
import jax.numpy as jnp
from jax import lax
from jax.experimental import pallas as pl
from jax.experimental.pallas import tpu as pltpu

N_EXPERTS = 4
EXPERTS_PER_SHARD = 2
CAP = 320


def kernel(x, assign, W1, W2):
    tokens, d_model = x.shape
    my_x = lax.axis_index("x")

    oh = (assign[:, None] == jnp.arange(N_EXPERTS, dtype=assign.dtype)[None, :]).astype(jnp.int32)
    pos = ((jnp.cumsum(oh, axis=0) - 1) * oh).sum(axis=1)
    ap_row = jnp.stack([assign.astype(jnp.int32), pos])
    ap_col = ap_row.T

    f32 = jnp.float32
    bf16 = jnp.bfloat16
    K = EXPERTS_PER_SHARD

    def body(x_ref, apr_ref, apc_ref, w1_ref, w2_ref,
             out_ref, xin, sendbuf, rescat, send_sems, recv_sems):
        mx = lax.axis_index("x")
        my = lax.axis_index("y")
        mz = lax.axis_index("z")
        peer = (1 - mx, my, mz)
        e_loc = [2 * mx + k for k in range(K)]
        e_out = [2 * (1 - mx) + k for k in range(K)]

        def make_p(e):
            iota = lax.broadcasted_iota(jnp.int32, (CAP, tokens), 0)
            sel = (apr_ref[0:1, :] == e) & (apr_ref[1:2, :] == iota)
            return sel.astype(bf16)

        def make_p2(e0, e1):
            iota = lax.broadcasted_iota(jnp.int32, (2 * CAP, tokens), 0)
            low = iota < CAP
            expert = jnp.where(low, e0, e1)
            rank = jnp.where(low, iota, iota - CAP)
            sel = (apr_ref[0:1, :] == expert) & (apr_ref[1:2, :] == rank)
            return sel.astype(bf16)

        def make_pt_all():
            c = lax.broadcasted_iota(jnp.int32, (tokens, 4 * CAP), 1)
            expert = jnp.where(
                c < CAP, e_loc[0],
                jnp.where(c < 2 * CAP, e_loc[1],
                          jnp.where(c < 3 * CAP, e_out[0], e_out[1])))
            rank = jnp.where(
                c < CAP, c,
                jnp.where(c < 2 * CAP, c - CAP,
                          jnp.where(c < 3 * CAP, c - 2 * CAP, c - 3 * CAP)))
            sel = (apc_ref[:, 0:1] == expert) & (apc_ref[:, 1:2] == rank)
            return sel.astype(bf16)

        xb = x_ref[...].astype(bf16)

        barrier_sem = pltpu.get_barrier_semaphore()
        pl.semaphore_signal(barrier_sem, inc=1, device_id=peer,
                            device_id_type=pl.DeviceIdType.MESH)
        pl.semaphore_wait(barrier_sem, 1)

        sendbuf[...] = jnp.dot(make_p2(e_out[0], e_out[1]), xb,
                               preferred_element_type=f32).astype(bf16)
        rdma_x = []
        for k in range(K):
            r = pltpu.make_async_remote_copy(
                src_ref=sendbuf.at[pl.ds(k * CAP, CAP), :],
                dst_ref=xin.at[pl.ds(k * CAP, CAP), :],
                send_sem=send_sems.at[k], recv_sem=recv_sems.at[k],
                device_id=peer, device_id_type=pl.DeviceIdType.MESH)
            r.start()
            rdma_x.append(r)

        xg = [jnp.dot(make_p(e_loc[k]), xb, preferred_element_type=f32)
              for k in range(K)]
        h = [jnp.maximum(jnp.dot(xg[k], w1_ref[k], preferred_element_type=f32), 0.0)
             for k in range(K)]
        res = [jnp.dot(h[k], w2_ref[k], preferred_element_type=f32)
               for k in range(K)]
        for k in range(K):
            rescat[pl.ds(k * CAP, CAP), :] = res[k].astype(bf16)

        rdma_r = []
        for k in range(K):
            rdma_x[k].wait_recv()
            xp = xin[pl.ds(k * CAP, CAP), :].astype(f32)
            hp = jnp.maximum(jnp.dot(xp, w1_ref[k], preferred_element_type=f32), 0.0)
            resp = jnp.dot(hp, w2_ref[k], preferred_element_type=f32)
            rdma_x[k].wait_send()
            sendbuf[pl.ds(k * CAP, CAP), :] = resp.astype(bf16)
            r = pltpu.make_async_remote_copy(
                src_ref=sendbuf.at[pl.ds(k * CAP, CAP), :],
                dst_ref=rescat.at[pl.ds((2 + k) * CAP, CAP), :],
                send_sem=send_sems.at[K + k],
                recv_sem=recv_sems.at[K + k],
                device_id=peer, device_id_type=pl.DeviceIdType.MESH)
            r.start()
            rdma_r.append(r)

        for r in rdma_r:
            r.wait_recv()
        out_ref[...] = jnp.dot(make_pt_all(), rescat[...],
                               preferred_element_type=f32)
        for r in rdma_r:
            r.wait_send()

    return pl.pallas_call(
        body,
        out_shape=jax.ShapeDtypeStruct((tokens, d_model), x.dtype),
        in_specs=[pl.BlockSpec(memory_space=pltpu.VMEM)] * 5,
        out_specs=pl.BlockSpec(memory_space=pltpu.VMEM),
        scratch_shapes=[
            pltpu.VMEM((2 * CAP, d_model), bf16),
            pltpu.VMEM((2 * CAP, d_model), bf16),
            pltpu.VMEM((4 * CAP, d_model), bf16),
            pltpu.SemaphoreType.DMA((2 * K,)),
            pltpu.SemaphoreType.DMA((2 * K,)),
        ],
        compiler_params=pltpu.CompilerParams(
            collective_id=0,
            vmem_limit_bytes=60 * 1024 * 1024,
        ),
    )(x, ap_row, ap_col, W1, W2)


# device time: 57671 ns/iter; 1.0469x vs baseline; 1.0426x over previous
import jax
import jax.numpy as jnp
from jax import lax
from jax.experimental import pallas as pl
from jax.experimental.pallas import tpu as pltpu

N_EXPERTS = 4
EXPERTS_PER_SHARD = 2
CAP = 320


def kernel(x, assign, W1, W2):
    tokens, d_model = x.shape
    my_x = lax.axis_index("x")

    oh = (assign[:, None] == jnp.arange(N_EXPERTS, dtype=assign.dtype)[None, :]).astype(jnp.int32)
    pos = ((jnp.cumsum(oh, axis=0) - 1) * oh).sum(axis=1)
    ap_row = jnp.stack([assign.astype(jnp.int32), pos])
    ap_col = ap_row.T

    f32 = jnp.float32
    bf16 = jnp.bfloat16

    def body(x_ref, apr_ref, apc_ref, w1_ref, w2_ref,
             out_ref, xin, resout, resb, send_sems, recv_sems):
        mx = lax.axis_index("x")
        my = lax.axis_index("y")
        mz = lax.axis_index("z")
        peer = (1 - mx, my, mz)

        def make_p(e):
            iota = lax.broadcasted_iota(jnp.int32, (CAP, tokens), 0)
            sel = (apr_ref[0:1, :] == e) & (apr_ref[1:2, :] == iota)
            return sel.astype(bf16)

        def make_pt(e):
            iota = lax.broadcasted_iota(jnp.int32, (tokens, CAP), 1)
            sel = (apc_ref[:, 0:1] == e) & (apc_ref[:, 1:2] == iota)
            return sel.astype(bf16)

        barrier_sem = pltpu.get_barrier_semaphore()
        pl.semaphore_signal(barrier_sem, inc=1, device_id=peer,
                            device_id_type=pl.DeviceIdType.MESH)
        pl.semaphore_wait(barrier_sem, 1)

        xb = x_ref[...].astype(bf16)

        rdma_x = []
        for k in range(EXPERTS_PER_SHARD):
            e_out = 2 * (1 - mx) + k
            resout[k] = jnp.dot(make_p(e_out), xb, preferred_element_type=f32).astype(bf16)
            r = pltpu.make_async_remote_copy(
                src_ref=resout.at[k], dst_ref=xin.at[k],
                send_sem=send_sems.at[k], recv_sem=recv_sems.at[k],
                device_id=peer, device_id_type=pl.DeviceIdType.MESH)
            r.start()
            rdma_x.append(r)

        e_locs = [2 * mx + k for k in range(EXPERTS_PER_SHARD)]
        xg = [jnp.dot(make_p(e), xb, preferred_element_type=f32) for e in e_locs]
        h = [jnp.maximum(jnp.dot(xg[k], w1_ref[k], preferred_element_type=f32), 0.0)
             for k in range(EXPERTS_PER_SHARD)]
        res = [jnp.dot(h[k], w2_ref[k], preferred_element_type=f32)
               for k in range(EXPERTS_PER_SHARD)]
        acc = jnp.dot(make_pt(e_locs[0]), res[0].astype(bf16), preferred_element_type=f32)
        acc = acc + jnp.dot(make_pt(e_locs[1]), res[1].astype(bf16), preferred_element_type=f32)
        out_ref[...] = acc

        rdma_r = []
        for k in range(EXPERTS_PER_SHARD):
            rdma_x[k].wait_send()
            rdma_x[k].wait_recv()
            xp = xin[k][...].astype(f32)
            h = jnp.maximum(jnp.dot(xp, w1_ref[k], preferred_element_type=f32), 0.0)
            resout[k] = jnp.dot(h, w2_ref[k], preferred_element_type=f32).astype(bf16)
            r = pltpu.make_async_remote_copy(
                src_ref=resout.at[k], dst_ref=resb.at[k],
                send_sem=send_sems.at[EXPERTS_PER_SHARD + k],
                recv_sem=recv_sems.at[EXPERTS_PER_SHARD + k],
                device_id=peer, device_id_type=pl.DeviceIdType.MESH)
            r.start()
            rdma_r.append(r)

        for k in range(EXPERTS_PER_SHARD):
            e_out = 2 * (1 - mx) + k
            rdma_r[k].wait_recv()
            out_ref[...] = out_ref[...] + jnp.dot(
                make_pt(e_out), resb[k], preferred_element_type=f32)
        for r in rdma_r:
            r.wait_send()

    return pl.pallas_call(
        body,
        out_shape=jax.ShapeDtypeStruct((tokens, d_model), x.dtype),
        in_specs=[pl.BlockSpec(memory_space=pltpu.VMEM)] * 5,
        out_specs=pl.BlockSpec(memory_space=pltpu.VMEM),
        scratch_shapes=[
            pltpu.VMEM((EXPERTS_PER_SHARD, CAP, d_model), bf16),
            pltpu.VMEM((EXPERTS_PER_SHARD, CAP, d_model), bf16),
            pltpu.VMEM((EXPERTS_PER_SHARD, CAP, d_model), bf16),
            pltpu.SemaphoreType.DMA((2 * EXPERTS_PER_SHARD,)),
            pltpu.SemaphoreType.DMA((2 * EXPERTS_PER_SHARD,)),
        ],
        compiler_params=pltpu.CompilerParams(
            collective_id=0,
            vmem_limit_bytes=60 * 1024 * 1024,
        ),
    )(x, ap_row, ap_col, W1, W2)
